# baseline (device time: 23781 ns/iter reference)
import jax
import jax.numpy as jnp
from jax import lax
from jax.experimental import pallas as pl
from jax.experimental.pallas import tpu as pltpu

N_DEV = 16
N_TOK = 512
D_IN = 256
D_OUT = 512
N_EXP = 32
EXP_PER_DEV = N_EXP // N_DEV
CHUNK = N_TOK // N_DEV
G = 2
CD = D_OUT // G


def kernel(x, router_W, route_idx, expert_W):
    def body(x_ref, rw_ref, idx_ref, ew_ref, out_ref,
             own_ref, comm_ref,
             send_sems1, send_sems2, recv_sems1, recv_sems2):
        my = lax.axis_index("i")

        barrier = pltpu.get_barrier_semaphore()
        for d in range(1, N_DEV):
            peer = lax.rem(my + d, N_DEV)
            pl.semaphore_signal(
                barrier, inc=1,
                device_id=(peer,), device_id_type=pl.DeviceIdType.MESH,
            )

        xf = x_ref[...]
        scores = jnp.dot(xf, rw_ref[...], preferred_element_type=jnp.float32)
        scores = scores - jnp.max(scores, axis=-1, keepdims=True)
        probs = jnp.exp(scores)
        probs = probs / jnp.sum(probs, axis=-1, keepdims=True)

        idx0 = idx_ref[:, 0:1]
        idx1 = idx_ref[:, 1:2]
        eid = lax.broadcasted_iota(jnp.int32, (N_TOK, N_EXP), 1)
        g0 = jnp.sum(jnp.where(eid == idx0, probs, 0.0), axis=-1, keepdims=True)
        g1 = jnp.sum(jnp.where(eid == idx1, probs, 0.0), axis=-1, keepdims=True)
        gs = g0 + g1
        w0 = g0 / gs
        w1 = g1 / gs

        xs = []
        for j in range(EXP_PER_DEV):
            ge = my * EXP_PER_DEV + j
            cj = jnp.where(idx0 == ge, w0, 0.0) + jnp.where(idx1 == ge, w1, 0.0)
            xs.append((xf * cj).astype(jnp.bfloat16))

        for g in range(G):
            acc = jnp.zeros((N_TOK, CD), jnp.float32)
            for j in range(EXP_PER_DEV):
                acc = acc + jnp.dot(
                    xs[j], ew_ref[j, :, g * CD:(g + 1) * CD].astype(jnp.bfloat16),
                    preferred_element_type=jnp.float32,
                )
            own_ref[:, g * CD:(g + 1) * CD] = acc.astype(jnp.bfloat16)

            if g == 0:
                pl.semaphore_wait(barrier, N_DEV - 1)

            for d in range(1, N_DEV):
                peer = lax.rem(my + d, N_DEV)
                rdma = pltpu.make_async_remote_copy(
                    src_ref=own_ref.at[pl.ds(peer * CHUNK, CHUNK),
                                       pl.ds(g * CD, CD)],
                    dst_ref=comm_ref.at[g, my],
                    send_sem=send_sems1.at[g, d - 1],
                    recv_sem=recv_sems1.at[g, my],
                    device_id=(peer,),
                    device_id_type=pl.DeviceIdType.MESH,
                )
                rdma.start()

        for g in range(G):
            red = own_ref[pl.ds(my * CHUNK, CHUNK),
                          pl.ds(g * CD, CD)].astype(jnp.float32)
            for d in range(1, N_DEV):
                src = lax.rem(my + d, N_DEV)
                recv = pltpu.make_async_remote_copy(
                    src_ref=comm_ref.at[g, 0],
                    dst_ref=comm_ref.at[g, src],
                    send_sem=send_sems1.at[g, 0],
                    recv_sem=recv_sems1.at[g, src],
                    device_id=(src,),
                    device_id_type=pl.DeviceIdType.MESH,
                )
                recv.wait_recv()
                red = red + comm_ref[g, src].astype(jnp.float32)
            out_ref[pl.ds(my * CHUNK, CHUNK),
                    pl.ds(g * CD, CD)] = red.astype(jnp.bfloat16)

            for d in range(1, N_DEV):
                peer = lax.rem(my + d, N_DEV)
                rdma = pltpu.make_async_remote_copy(
                    src_ref=out_ref.at[pl.ds(my * CHUNK, CHUNK),
                                       pl.ds(g * CD, CD)],
                    dst_ref=out_ref.at[pl.ds(my * CHUNK, CHUNK),
                                       pl.ds(g * CD, CD)],
                    send_sem=send_sems2.at[g, d - 1],
                    recv_sem=recv_sems2.at[g, my],
                    device_id=(peer,),
                    device_id_type=pl.DeviceIdType.MESH,
                )
                rdma.start()

        for g in range(G):
            for d in range(1, N_DEV):
                peer = lax.rem(my + d, N_DEV)
                send = pltpu.make_async_remote_copy(
                    src_ref=comm_ref.at[g, 0],
                    dst_ref=comm_ref.at[g, 0],
                    send_sem=send_sems1.at[g, d - 1],
                    recv_sem=recv_sems1.at[g, 0],
                    device_id=(peer,),
                    device_id_type=pl.DeviceIdType.MESH,
                )
                send.wait_send()

        for g in range(G):
            for d in range(1, N_DEV):
                src = lax.rem(my + d, N_DEV)
                recv = pltpu.make_async_remote_copy(
                    src_ref=comm_ref.at[g, 0],
                    dst_ref=out_ref.at[pl.ds(src * CHUNK, CHUNK),
                                       pl.ds(g * CD, CD)],
                    send_sem=send_sems2.at[g, 0],
                    recv_sem=recv_sems2.at[g, src],
                    device_id=(src,),
                    device_id_type=pl.DeviceIdType.MESH,
                )
                recv.wait_recv()

        for g in range(G):
            for d in range(1, N_DEV):
                peer = lax.rem(my + d, N_DEV)
                send = pltpu.make_async_remote_copy(
                    src_ref=comm_ref.at[g, 0],
                    dst_ref=comm_ref.at[g, 0],
                    send_sem=send_sems2.at[g, d - 1],
                    recv_sem=recv_sems2.at[g, 0],
                    device_id=(peer,),
                    device_id_type=pl.DeviceIdType.MESH,
                )
                send.wait_send()

    return pl.pallas_call(
        body,
        out_shape=jax.ShapeDtypeStruct((N_TOK, D_OUT), jnp.bfloat16),
        in_specs=[
            pl.BlockSpec(memory_space=pltpu.VMEM),
            pl.BlockSpec(memory_space=pltpu.VMEM),
            pl.BlockSpec(memory_space=pltpu.VMEM),
            pl.BlockSpec(memory_space=pltpu.VMEM),
        ],
        out_specs=pl.BlockSpec(memory_space=pltpu.VMEM),
        scratch_shapes=[
            pltpu.VMEM((N_TOK, D_OUT), jnp.bfloat16),
            pltpu.VMEM((G, N_DEV, CHUNK, CD), jnp.bfloat16),
            pltpu.SemaphoreType.DMA((G, N_DEV - 1)),
            pltpu.SemaphoreType.DMA((G, N_DEV - 1)),
            pltpu.SemaphoreType.DMA((G, N_DEV)),
            pltpu.SemaphoreType.DMA((G, N_DEV)),
        ],
        compiler_params=pltpu.CompilerParams(collective_id=0),
    )(x, router_W, route_idx, expert_W)


# device time: 22607 ns/iter; 1.0519x vs baseline; 1.0519x over previous
import jax
import jax.numpy as jnp
from jax import lax
from jax.experimental import pallas as pl
from jax.experimental.pallas import tpu as pltpu

N_DEV = 16
N_TOK = 512
D_IN = 256
D_OUT = 512
N_EXP = 32
EXP_PER_DEV = N_EXP // N_DEV
CHUNK = N_TOK // N_DEV


def kernel(x, router_W, route_idx, expert_W):
    expert_W = expert_W.astype(jnp.bfloat16)
    def body(x_ref, rw_ref, idx_ref, ew_ref, out_ref,
             own_ref, comm_ref,
             send_sems1, send_sems2, recv_sems1, recv_sems2):
        my = lax.axis_index("i")

        barrier = pltpu.get_barrier_semaphore()
        for d in range(1, N_DEV):
            peer = lax.rem(my + d, N_DEV)
            pl.semaphore_signal(
                barrier, inc=1,
                device_id=(peer,), device_id_type=pl.DeviceIdType.MESH,
            )

        xf = x_ref[...]
        scores = jnp.dot(xf, rw_ref[...], preferred_element_type=jnp.float32)
        scores = scores - jnp.max(scores, axis=-1, keepdims=True)
        probs = jnp.exp(scores)
        probs = probs / jnp.sum(probs, axis=-1, keepdims=True)

        idx0 = idx_ref[:, 0:1]
        idx1 = idx_ref[:, 1:2]
        eid = lax.broadcasted_iota(jnp.int32, (N_TOK, N_EXP), 1)
        g0 = jnp.sum(jnp.where(eid == idx0, probs, 0.0), axis=-1, keepdims=True)
        g1 = jnp.sum(jnp.where(eid == idx1, probs, 0.0), axis=-1, keepdims=True)
        gs = g0 + g1
        w0 = g0 / gs
        w1 = g1 / gs

        parts = []
        for j in range(EXP_PER_DEV):
            ge = my * EXP_PER_DEV + j
            cj = jnp.where(idx0 == ge, w0, 0.0) + jnp.where(idx1 == ge, w1, 0.0)
            xs = (xf * cj).astype(jnp.bfloat16)
            parts.append(jnp.dot(xs, ew_ref[j],
                                 preferred_element_type=jnp.float32))
        acc = parts[0] + parts[1]
        own_ref[...] = acc.astype(jnp.bfloat16)

        pl.semaphore_wait(barrier, N_DEV - 1)

        for d in range(1, N_DEV):
            peer = lax.rem(my + d, N_DEV)
            rdma = pltpu.make_async_remote_copy(
                src_ref=own_ref.at[pl.ds(peer * CHUNK, CHUNK)],
                dst_ref=comm_ref.at[my],
                send_sem=send_sems1.at[d - 1],
                recv_sem=recv_sems1.at[my],
                device_id=(peer,),
                device_id_type=pl.DeviceIdType.MESH,
            )
            rdma.start()

        red = own_ref[pl.ds(my * CHUNK, CHUNK)].astype(jnp.float32)
        for d in range(1, N_DEV):
            src = lax.rem(my + d, N_DEV)
            recv = pltpu.make_async_remote_copy(
                src_ref=own_ref.at[pl.ds(0, CHUNK)],
                dst_ref=comm_ref.at[src],
                send_sem=send_sems1.at[0],
                recv_sem=recv_sems1.at[src],
                device_id=(src,),
                device_id_type=pl.DeviceIdType.MESH,
            )
            recv.wait_recv()
            red = red + comm_ref[src].astype(jnp.float32)
        out_ref[pl.ds(my * CHUNK, CHUNK), :] = red.astype(jnp.bfloat16)

        for d in range(1, N_DEV):
            peer = lax.rem(my + d, N_DEV)
            rdma = pltpu.make_async_remote_copy(
                src_ref=out_ref.at[pl.ds(my * CHUNK, CHUNK)],
                dst_ref=out_ref.at[pl.ds(my * CHUNK, CHUNK)],
                send_sem=send_sems2.at[d - 1],
                recv_sem=recv_sems2.at[my],
                device_id=(peer,),
                device_id_type=pl.DeviceIdType.MESH,
            )
            rdma.start()

        for d in range(1, N_DEV):
            peer = lax.rem(my + d, N_DEV)
            send = pltpu.make_async_remote_copy(
                src_ref=own_ref.at[pl.ds(0, CHUNK)],
                dst_ref=comm_ref.at[0],
                send_sem=send_sems1.at[d - 1],
                recv_sem=recv_sems1.at[0],
                device_id=(peer,),
                device_id_type=pl.DeviceIdType.MESH,
            )
            send.wait_send()

        for d in range(1, N_DEV):
            src = lax.rem(my + d, N_DEV)
            recv = pltpu.make_async_remote_copy(
                src_ref=out_ref.at[pl.ds(0, CHUNK)],
                dst_ref=out_ref.at[pl.ds(src * CHUNK, CHUNK)],
                send_sem=send_sems2.at[0],
                recv_sem=recv_sems2.at[src],
                device_id=(src,),
                device_id_type=pl.DeviceIdType.MESH,
            )
            recv.wait_recv()

        for d in range(1, N_DEV):
            peer = lax.rem(my + d, N_DEV)
            send = pltpu.make_async_remote_copy(
                src_ref=out_ref.at[pl.ds(0, CHUNK)],
                dst_ref=comm_ref.at[0],
                send_sem=send_sems2.at[d - 1],
                recv_sem=recv_sems1.at[0],
                device_id=(peer,),
                device_id_type=pl.DeviceIdType.MESH,
            )
            send.wait_send()

    return pl.pallas_call(
        body,
        out_shape=jax.ShapeDtypeStruct((N_TOK, D_OUT), jnp.bfloat16),
        in_specs=[
            pl.BlockSpec(memory_space=pltpu.VMEM),
            pl.BlockSpec(memory_space=pltpu.VMEM),
            pl.BlockSpec(memory_space=pltpu.VMEM),
            pl.BlockSpec(memory_space=pltpu.VMEM),
        ],
        out_specs=pl.BlockSpec(memory_space=pltpu.VMEM),
        scratch_shapes=[
            pltpu.VMEM((N_TOK, D_OUT), jnp.bfloat16),
            pltpu.VMEM((N_DEV, CHUNK, D_OUT), jnp.bfloat16),
            pltpu.SemaphoreType.DMA((N_DEV - 1,)),
            pltpu.SemaphoreType.DMA((N_DEV - 1,)),
            pltpu.SemaphoreType.DMA((N_DEV,)),
            pltpu.SemaphoreType.DMA((N_DEV,)),
        ],
        compiler_params=pltpu.CompilerParams(collective_id=0),
    )(x, router_W, route_idx, expert_W)
